# Initial kernel scaffold; baseline (speedup 1.0000x reference)
import functools

import jax
import jax.numpy as jnp
from jax import lax
from jax.experimental import pallas as pl
from jax.experimental.pallas import tpu as pltpu
from jax.experimental.pallas import tpu_sc as plsc

N = 100000
E = 3200000
NC = 2
NS = 16
NW = NC * NS
PER_TILE = E // NW
WIN = 4000
NWIN = PER_TILE // WIN
ROWS_PER_TILE = N // NS

_SC_MESH = plsc.VectorSubcoreMesh(core_axis_name="c", subcore_axis_name="s")


def _sc_hist_body(dst_hbm, zeros_hbm, ones_hbm, out_hbm, idx_v, ones_v,
                  deg_sh, sem):
    c = lax.axis_index("c")
    s = lax.axis_index("s")
    wid = c * NS + s
    row0 = s * ROWS_PER_TILE
    pltpu.sync_copy(ones_hbm, ones_v)
    pltpu.sync_copy(zeros_hbm.at[pl.ds(row0, ROWS_PER_TILE), :],
                    deg_sh.at[pl.ds(row0, ROWS_PER_TILE), :])
    plsc.subcore_barrier()
    base_e = wid * PER_TILE

    def body(w, carry):
        pltpu.sync_copy(dst_hbm.at[pl.ds(base_e + w * WIN, WIN)], idx_v)
        pltpu.sync_copy(ones_v, deg_sh.at[idx_v], add=True)
        return carry

    lax.fori_loop(0, NWIN, body, 0)
    plsc.subcore_barrier()
    pltpu.sync_copy(deg_sh.at[pl.ds(row0, ROWS_PER_TILE), :],
                    out_hbm.at[c, pl.ds(row0, ROWS_PER_TILE), :])


_sc_hist = pl.kernel(
    _sc_hist_body,
    out_type=jax.ShapeDtypeStruct((NC, N, 4), jnp.float32),
    mesh=_SC_MESH,
    scratch_types=[
        pltpu.VMEM((WIN,), jnp.int32),
        pltpu.VMEM((WIN, 4), jnp.float32),
        pltpu.VMEM_SHARED((N, 4), jnp.float32),
        pltpu.SemaphoreType.DMA,
    ],
)


def _sc_msg_body(src_hbm, dst_hbm, p_hbm, zeros_hbm, out_hbm, src_v, dst_v,
                 rows_v, p_sh, acc_sh, sem):
    c = lax.axis_index("c")
    s = lax.axis_index("s")
    wid = c * NS + s
    row0 = s * ROWS_PER_TILE
    pltpu.sync_copy(p_hbm.at[pl.ds(row0, ROWS_PER_TILE), :],
                    p_sh.at[pl.ds(row0, ROWS_PER_TILE), :])
    pltpu.sync_copy(zeros_hbm.at[pl.ds(row0, ROWS_PER_TILE), :],
                    acc_sh.at[pl.ds(row0, ROWS_PER_TILE), :])
    plsc.subcore_barrier()
    base_e = wid * PER_TILE

    def body(w, carry):
        pltpu.sync_copy(src_hbm.at[pl.ds(base_e + w * WIN, WIN)], src_v)
        pltpu.sync_copy(dst_hbm.at[pl.ds(base_e + w * WIN, WIN)], dst_v)
        pltpu.async_copy(p_sh.at[src_v], rows_v, sem).wait()
        pltpu.sync_copy(rows_v, acc_sh.at[dst_v], add=True)
        return carry

    lax.fori_loop(0, NWIN, body, 0)
    plsc.subcore_barrier()
    pltpu.sync_copy(acc_sh.at[pl.ds(row0, ROWS_PER_TILE), :],
                    out_hbm.at[c, pl.ds(row0, ROWS_PER_TILE), :])


_sc_msg = pl.kernel(
    _sc_msg_body,
    out_type=jax.ShapeDtypeStruct((NC, N, 4), jnp.float32),
    mesh=_SC_MESH,
    scratch_types=[
        pltpu.VMEM((WIN,), jnp.int32),
        pltpu.VMEM((WIN,), jnp.int32),
        pltpu.VMEM((WIN, 4), jnp.float32),
        pltpu.VMEM_SHARED((N, 4), jnp.float32),
        pltpu.VMEM_SHARED((N, 4), jnp.float32),
        pltpu.SemaphoreType.DMA,
    ],
)


def _tc_pre_body(x_ref, dega_ref, degb_ref, p_ref, dinv_ref):
    deg = dega_ref[...] + degb_ref[...] + 1.0
    dinv = lax.rsqrt(deg)
    dinv_ref[...] = dinv
    p_ref[...] = x_ref[...] * dinv


def _tc_pre(x_flat, dega, degb):
    return pl.pallas_call(
        _tc_pre_body,
        out_shape=(
            jax.ShapeDtypeStruct((N // 32, 128), jnp.float32),
            jax.ShapeDtypeStruct((N // 32, 128), jnp.float32),
        ),
    )(x_flat, dega, degb)


def _tc_gcn_body(sa_ref, sb_ref, p_ref, dinv_ref, bigw_ref, bias_ref,
                 out_ref):
    u = dinv_ref[...] * (sa_ref[...] + sb_ref[...] + p_ref[...])
    t = jnp.dot(u, bigw_ref[...], preferred_element_type=jnp.float32)
    out_ref[...] = jnp.maximum(t + bias_ref[...], 0.0)


def _tc_gcn(sa, sb, p_flat, dinv_flat, bigw, bias_flat):
    return pl.pallas_call(
        _tc_gcn_body,
        out_shape=jax.ShapeDtypeStruct((N // 32, 512), jnp.float32),
    )(sa, sb, p_flat, dinv_flat, bigw, bias_flat)


_CB = 1024
_LPAD = 100352
_NBLK = _LPAD // _CB


def _tc_conv_body(zc_ref, zl_ref, zr_ref, w0_ref, w1_ref, w2_ref, cb_ref,
                  fcw_ref, fcb_ref, out_ref, acc_ref):
    i = pl.program_id(0)

    @pl.when(i == 0)
    def _init():
        acc_ref[...] = jnp.zeros_like(acc_ref)

    z = (jnp.dot(w0_ref[...], zl_ref[...], preferred_element_type=jnp.float32)
         + jnp.dot(w1_ref[...], zc_ref[...], preferred_element_type=jnp.float32)
         + jnp.dot(w2_ref[...], zr_ref[...], preferred_element_type=jnp.float32)
         + cb_ref[...])
    z = jnp.maximum(z, 0.0)
    col = i * _CB + lax.broadcasted_iota(jnp.int32, (32, _CB), 1)
    acc_ref[...] += jnp.where(col < N, z, 0.0)

    @pl.when(i == _NBLK - 1)
    def _fin():
        m = jnp.sum(acc_ref[...], axis=1, keepdims=True) * (1.0 / N)
        val = jnp.sum(m * fcw_ref[...]) + fcb_ref[0, 0]
        out_ref[0, 0] = 1.0 / (1.0 + jnp.exp(-val))


def _tc_conv(zc, zl, zr, w0, w1, w2, cb2, fcw2, fcb2):
    return pl.pallas_call(
        _tc_conv_body,
        grid=(_NBLK,),
        in_specs=[
            pl.BlockSpec((16, _CB), lambda i: (0, i)),
            pl.BlockSpec((16, _CB), lambda i: (0, i)),
            pl.BlockSpec((16, _CB), lambda i: (0, i)),
            pl.BlockSpec((32, 16), lambda i: (0, 0)),
            pl.BlockSpec((32, 16), lambda i: (0, 0)),
            pl.BlockSpec((32, 16), lambda i: (0, 0)),
            pl.BlockSpec((32, 1), lambda i: (0, 0)),
            pl.BlockSpec((32, 1), lambda i: (0, 0)),
            pl.BlockSpec((1, 1), lambda i: (0, 0)),
        ],
        out_specs=pl.BlockSpec((1, 1), lambda i: (0, 0)),
        out_shape=jax.ShapeDtypeStruct((1, 1), jnp.float32),
        scratch_shapes=[pltpu.VMEM((32, _CB), jnp.float32)],
    )(zc, zl, zr, w0, w1, w2, cb2, fcw2, fcb2)


def kernel(x, edge_index, W_gcn, b_gcn, conv_w, conv_b, fc_w, fc_b):
    src = edge_index[0]
    dst = edge_index[1]
    zeros4 = jnp.zeros((N, 4), jnp.float32)
    ones4 = jnp.ones((WIN, 4), jnp.float32)

    deg4 = _sc_hist(dst, zeros4, ones4)
    dega = deg4[0].reshape(N // 32, 128)
    degb = deg4[1].reshape(N // 32, 128)
    x_flat = x.reshape(N // 32, 128)

    p_flat, dinv_flat = _tc_pre(x_flat, dega, degb)

    p4 = p_flat.reshape(N, 4)
    s4 = _sc_msg(src, dst, p4, zeros4)
    sa = s4[0].reshape(N // 32, 128)
    sb = s4[1].reshape(N // 32, 128)

    bigw = jnp.kron(jnp.eye(32, dtype=jnp.float32), W_gcn)
    bias_flat = jnp.tile(b_gcn, 32).reshape(1, 512)
    out_flat = _tc_gcn(sa, sb, p_flat, dinv_flat, bigw, bias_flat)

    z = out_flat.reshape(16, N)
    zc = jnp.pad(z, ((0, 0), (0, _LPAD - N)))
    zl = jnp.pad(z[:, :N - 1], ((0, 0), (1, _LPAD - N)))
    zr = jnp.pad(z[:, 1:], ((0, 0), (0, _LPAD - N + 1)))

    w0 = conv_w[:, :, 0]
    w1 = conv_w[:, :, 1]
    w2 = conv_w[:, :, 2]
    cb2 = conv_b.reshape(32, 1)
    fcw2 = fc_w.reshape(32, 1)
    fcb2 = fc_b.reshape(1, 1)

    y = _tc_conv(zc, zl, zr, w0, w1, w2, cb2, fcw2, fcb2)
    return y.reshape(-1)


# speedup vs baseline: 71.5120x; 71.5120x over previous
import functools

import jax
import jax.numpy as jnp
from jax import lax
from jax.experimental import pallas as pl
from jax.experimental.pallas import tpu as pltpu
from jax.experimental.pallas import tpu_sc as plsc

N = 100000
NP = 100096
E = 3200000
NC = 2
NS = 16
NW = NC * NS
PER_TILE = E // NW
WIN = 4000
NWIN = PER_TILE // WIN
ROWS_PER_TILE = NP // NS

_SC_MESH = plsc.VectorSubcoreMesh(core_axis_name="c", subcore_axis_name="s")


def _sc_hist_body(dst_hbm, zeros_hbm, ones_hbm, out_hbm, idx_v, ones_v,
                  deg_sh, sem):
    c = lax.axis_index("c")
    s = lax.axis_index("s")
    wid = c * NS + s
    row0 = s * ROWS_PER_TILE
    pltpu.sync_copy(ones_hbm, ones_v)
    pltpu.sync_copy(zeros_hbm.at[pl.ds(row0, ROWS_PER_TILE), :],
                    deg_sh.at[pl.ds(row0, ROWS_PER_TILE), :])
    plsc.subcore_barrier()
    base_e = wid * PER_TILE

    def body(w, carry):
        pltpu.sync_copy(dst_hbm.at[pl.ds(base_e + w * WIN, WIN)], idx_v)
        pltpu.sync_copy(ones_v, deg_sh.at[idx_v], add=True)
        return carry

    lax.fori_loop(0, NWIN, body, 0)
    plsc.subcore_barrier()
    pltpu.sync_copy(deg_sh.at[pl.ds(row0, ROWS_PER_TILE), :],
                    out_hbm.at[c, pl.ds(row0, ROWS_PER_TILE), :])


_sc_hist = pl.kernel(
    _sc_hist_body,
    out_type=jax.ShapeDtypeStruct((NC, NP, 4), jnp.float32),
    mesh=_SC_MESH,
    compiler_params=pltpu.CompilerParams(use_tc_tiling_on_sc=False),
    scratch_types=[
        pltpu.VMEM((WIN,), jnp.int32),
        pltpu.VMEM((WIN, 4), jnp.float32),
        pltpu.VMEM_SHARED((NP, 4), jnp.float32),
        pltpu.SemaphoreType.DMA,
    ],
)


def _sc_msg_body(src_hbm, dst_hbm, p_hbm, zeros_hbm, out_hbm, src_v, dst_v,
                 rows_v, p_sh, acc_sh, sem):
    c = lax.axis_index("c")
    s = lax.axis_index("s")
    wid = c * NS + s
    row0 = s * ROWS_PER_TILE
    pltpu.sync_copy(p_hbm.at[pl.ds(row0, ROWS_PER_TILE), :],
                    p_sh.at[pl.ds(row0, ROWS_PER_TILE), :])
    pltpu.sync_copy(zeros_hbm.at[pl.ds(row0, ROWS_PER_TILE), :],
                    acc_sh.at[pl.ds(row0, ROWS_PER_TILE), :])
    plsc.subcore_barrier()
    base_e = wid * PER_TILE

    def body(w, carry):
        pltpu.sync_copy(src_hbm.at[pl.ds(base_e + w * WIN, WIN)], src_v)
        pltpu.sync_copy(dst_hbm.at[pl.ds(base_e + w * WIN, WIN)], dst_v)
        pltpu.async_copy(p_sh.at[src_v], rows_v, sem).wait()
        pltpu.sync_copy(rows_v, acc_sh.at[dst_v], add=True)
        return carry

    lax.fori_loop(0, NWIN, body, 0)
    plsc.subcore_barrier()
    pltpu.sync_copy(acc_sh.at[pl.ds(row0, ROWS_PER_TILE), :],
                    out_hbm.at[c, pl.ds(row0, ROWS_PER_TILE), :])


_sc_msg = pl.kernel(
    _sc_msg_body,
    out_type=jax.ShapeDtypeStruct((NC, NP, 4), jnp.float32),
    mesh=_SC_MESH,
    compiler_params=pltpu.CompilerParams(use_tc_tiling_on_sc=False),
    scratch_types=[
        pltpu.VMEM((WIN,), jnp.int32),
        pltpu.VMEM((WIN,), jnp.int32),
        pltpu.VMEM((WIN, 4), jnp.float32),
        pltpu.VMEM_SHARED((NP, 4), jnp.float32),
        pltpu.VMEM_SHARED((NP, 4), jnp.float32),
        pltpu.SemaphoreType.DMA,
    ],
)


def _tc_pre_body(x_ref, dega_ref, degb_ref, p_ref, dinv_ref):
    deg = dega_ref[...] + degb_ref[...] + 1.0
    dinv = lax.rsqrt(deg)
    dinv_ref[...] = dinv
    p_ref[...] = x_ref[...] * dinv


def _tc_pre(x_flat, dega, degb):
    return pl.pallas_call(
        _tc_pre_body,
        out_shape=(
            jax.ShapeDtypeStruct((NP // 32, 128), jnp.float32),
            jax.ShapeDtypeStruct((NP // 32, 128), jnp.float32),
        ),
    )(x_flat, dega, degb)


def _tc_gcn_body(sa_ref, sb_ref, p_ref, dinv_ref, bigw_ref, bias_ref,
                 out_ref):
    u = dinv_ref[...] * (sa_ref[...] + sb_ref[...] + p_ref[...])
    t = jnp.dot(u, bigw_ref[...], preferred_element_type=jnp.float32)
    out_ref[...] = jnp.maximum(t + bias_ref[...], 0.0)


def _tc_gcn(sa, sb, p_flat, dinv_flat, bigw, bias_flat):
    return pl.pallas_call(
        _tc_gcn_body,
        out_shape=jax.ShapeDtypeStruct((NP // 32, 512), jnp.float32),
    )(sa, sb, p_flat, dinv_flat, bigw, bias_flat)


_CB = 1024
_LPAD = 100352
_NBLK = _LPAD // _CB


def _tc_conv_body(zc_ref, zl_ref, zr_ref, w0_ref, w1_ref, w2_ref, cb_ref,
                  fcw_ref, fcb_ref, out_ref, acc_ref):
    i = pl.program_id(0)

    @pl.when(i == 0)
    def _init():
        acc_ref[...] = jnp.zeros_like(acc_ref)

    z = (jnp.dot(w0_ref[...], zl_ref[...], preferred_element_type=jnp.float32)
         + jnp.dot(w1_ref[...], zc_ref[...], preferred_element_type=jnp.float32)
         + jnp.dot(w2_ref[...], zr_ref[...], preferred_element_type=jnp.float32)
         + cb_ref[...])
    z = jnp.maximum(z, 0.0)
    col = i * _CB + lax.broadcasted_iota(jnp.int32, (32, _CB), 1)
    acc_ref[...] += jnp.where(col < N, z, 0.0)

    @pl.when(i == _NBLK - 1)
    def _fin():
        m = jnp.sum(acc_ref[...], axis=1, keepdims=True) * (1.0 / N)
        val = (jnp.sum(m * fcw_ref[...], keepdims=True).reshape(1, 1)
               + fcb_ref[...])
        out_ref[...] = 1.0 / (1.0 + jnp.exp(-val))


def _tc_conv(zc, zl, zr, w0, w1, w2, cb2, fcw2, fcb2):
    return pl.pallas_call(
        _tc_conv_body,
        grid=(_NBLK,),
        in_specs=[
            pl.BlockSpec((16, _CB), lambda i: (0, i)),
            pl.BlockSpec((16, _CB), lambda i: (0, i)),
            pl.BlockSpec((16, _CB), lambda i: (0, i)),
            pl.BlockSpec((32, 16), lambda i: (0, 0)),
            pl.BlockSpec((32, 16), lambda i: (0, 0)),
            pl.BlockSpec((32, 16), lambda i: (0, 0)),
            pl.BlockSpec((32, 1), lambda i: (0, 0)),
            pl.BlockSpec((32, 1), lambda i: (0, 0)),
            pl.BlockSpec((1, 1), lambda i: (0, 0)),
        ],
        out_specs=pl.BlockSpec((1, 1), lambda i: (0, 0)),
        out_shape=jax.ShapeDtypeStruct((1, 1), jnp.float32),
        scratch_shapes=[pltpu.VMEM((32, _CB), jnp.float32)],
    )(zc, zl, zr, w0, w1, w2, cb2, fcw2, fcb2)


def kernel(x, edge_index, W_gcn, b_gcn, conv_w, conv_b, fc_w, fc_b):
    src = edge_index[0]
    dst = edge_index[1]
    zeros4 = jnp.zeros((NP, 4), jnp.float32)
    ones4 = jnp.ones((WIN, 4), jnp.float32)

    deg4 = _sc_hist(dst, zeros4, ones4)
    dega = deg4[0].reshape(NP // 32, 128)
    degb = deg4[1].reshape(NP // 32, 128)
    x_flat = jnp.pad(x, ((0, NP - N), (0, 0))).reshape(NP // 32, 128)

    p_flat, dinv_flat = _tc_pre(x_flat, dega, degb)

    p4 = p_flat.reshape(NP, 4)
    s4 = _sc_msg(src, dst, p4, zeros4)
    sa = s4[0].reshape(NP // 32, 128)
    sb = s4[1].reshape(NP // 32, 128)

    bigw = jnp.kron(jnp.eye(32, dtype=jnp.float32), W_gcn)
    bias_flat = jnp.tile(b_gcn, 32).reshape(1, 512)
    out_flat = _tc_gcn(sa, sb, p_flat, dinv_flat, bigw, bias_flat)

    z = out_flat[:N // 32].reshape(16, N)
    zc = jnp.pad(z, ((0, 0), (0, _LPAD - N)))
    zl = jnp.pad(z[:, :N - 1], ((0, 0), (1, _LPAD - N)))
    zr = jnp.pad(z[:, 1:], ((0, 0), (0, _LPAD - N + 1)))

    w0 = conv_w[:, :, 0]
    w1 = conv_w[:, :, 1]
    w2 = conv_w[:, :, 2]
    cb2 = conv_b.reshape(32, 1)
    fcw2 = fc_w.reshape(32, 1)
    fcb2 = fc_b.reshape(1, 1)

    y = _tc_conv(zc, zl, zr, w0, w1, w2, cb2, fcw2, fcb2)
    return y.reshape(-1)


# speedup vs baseline: 86.1099x; 1.2041x over previous
import jax
import jax.numpy as jnp
from jax import lax
from jax.experimental import pallas as pl
from jax.experimental.pallas import tpu as pltpu
from jax.experimental.pallas import tpu_sc as plsc

N = 100000
NP = 102400
E = 3200000
NC = 2
NS = 16
NW = NC * NS
PER_TILE = E // NW
WIN = 4000
NWIN = PER_TILE // WIN
RPT = NP // NS
FR = RPT * 4 // 128
FROWS = NP * 4 // 128
ZR = 40

_SC_MESH = plsc.VectorSubcoreMesh(core_axis_name="c", subcore_axis_name="s")
_SC_PARAMS = pltpu.CompilerParams(use_tc_tiling_on_sc=False)


def _sc_hist_body(dst_hbm, ones_hbm, zeros_hbm, out_hbm, idx_v, ones_v,
                  deg_sh, sem):
    c = lax.axis_index("c")
    s = lax.axis_index("s")
    wid = c * NS + s
    pltpu.sync_copy(ones_hbm, ones_v)
    pltpu.sync_copy(zeros_hbm, deg_sh.at[pl.ds(s * RPT, RPT), :])
    plsc.subcore_barrier()
    base_e = wid * PER_TILE

    def body(w, carry):
        pltpu.sync_copy(dst_hbm.at[pl.ds(base_e + w * WIN, WIN)], idx_v)
        pltpu.sync_copy(ones_v, deg_sh.at[idx_v], add=True)
        return carry

    lax.fori_loop(0, NWIN, body, 0)
    plsc.subcore_barrier()
    pltpu.sync_copy(deg_sh.at[pl.ds(s * RPT, RPT), :],
                    out_hbm.at[c, pl.ds(s * RPT, RPT), :])


_sc_hist = pl.kernel(
    _sc_hist_body,
    out_type=jax.ShapeDtypeStruct((NC, NP, 4), jnp.float32),
    mesh=_SC_MESH,
    compiler_params=_SC_PARAMS,
    scratch_types=[
        pltpu.VMEM((WIN,), jnp.int32),
        pltpu.VMEM((WIN, 4), jnp.float32),
        pltpu.VMEM_SHARED((NP, 4), jnp.float32),
        pltpu.SemaphoreType.DMA,
    ],
)


def _sc_msg_body(src_hbm, dst_hbm, p_hbm, zeros_hbm, out_hbm, src_v, dst_v,
                 rows_v, p_sh, acc_sh, sem):
    c = lax.axis_index("c")
    s = lax.axis_index("s")
    wid = c * NS + s
    pltpu.sync_copy(p_hbm.at[pl.ds(s * RPT, RPT), :],
                    p_sh.at[pl.ds(s * RPT, RPT), :])
    pltpu.sync_copy(zeros_hbm, acc_sh.at[pl.ds(s * RPT, RPT), :])
    plsc.subcore_barrier()
    base_e = wid * PER_TILE

    def body(w, carry):
        pltpu.sync_copy(src_hbm.at[pl.ds(base_e + w * WIN, WIN)], src_v)
        pltpu.sync_copy(dst_hbm.at[pl.ds(base_e + w * WIN, WIN)], dst_v)
        pltpu.async_copy(p_sh.at[src_v], rows_v, sem).wait()
        pltpu.sync_copy(rows_v, acc_sh.at[dst_v], add=True)
        return carry

    lax.fori_loop(0, NWIN, body, 0)
    plsc.subcore_barrier()
    pltpu.sync_copy(acc_sh.at[pl.ds(s * RPT, RPT), :],
                    out_hbm.at[c, pl.ds(s * RPT, RPT), :])


_sc_msg = pl.kernel(
    _sc_msg_body,
    out_type=jax.ShapeDtypeStruct((NC, NP, 4), jnp.float32),
    mesh=_SC_MESH,
    compiler_params=_SC_PARAMS,
    scratch_types=[
        pltpu.VMEM((WIN,), jnp.int32),
        pltpu.VMEM((WIN,), jnp.int32),
        pltpu.VMEM((WIN, 4), jnp.float32),
        pltpu.VMEM_SHARED((NP, 4), jnp.float32),
        pltpu.VMEM_SHARED((NP, 4), jnp.float32),
        pltpu.SemaphoreType.DMA,
    ],
)


def _tc_pre_body(x_ref, deg_ref, p_ref, dinv_ref):
    deg = deg_ref[0] + deg_ref[1] + 1.0
    dinv = lax.rsqrt(deg)
    dinv_ref[...] = dinv
    p_ref[...] = x_ref[...] * dinv


def _tc_pre(x_flat, deg4):
    return pl.pallas_call(
        _tc_pre_body,
        out_shape=(
            jax.ShapeDtypeStruct((FROWS, 128), jnp.float32),
            jax.ShapeDtypeStruct((FROWS, 128), jnp.float32),
        ),
    )(x_flat, deg4)


def _tc_gcn_body(s4_ref, p_ref, dinv_ref, bigw_ref, bias_ref, out_ref):
    u = dinv_ref[...] * (s4_ref[0] + s4_ref[1] + p_ref[...])
    t = jnp.dot(u, bigw_ref[...], preferred_element_type=jnp.float32)
    out_ref[...] = jnp.maximum(t + bias_ref[...], 0.0)


def _tc_gcn(s4, p_flat, dinv_flat, bigw, bias_flat):
    return pl.pallas_call(
        _tc_gcn_body,
        out_shape=jax.ShapeDtypeStruct((FROWS, 512), jnp.float32),
    )(s4, p_flat, dinv_flat, bigw, bias_flat)


_CB = 1024
_LPAD = 100352
_NBLK = _LPAD // _CB


def _tc_conv_body(zc_ref, zl_ref, zr_ref, w0_ref, w1_ref, w2_ref, cb_ref,
                  fcw_ref, fcb_ref, out_ref, acc_ref):
    i = pl.program_id(0)

    @pl.when(i == 0)
    def _init():
        acc_ref[...] = jnp.zeros_like(acc_ref)

    z = (jnp.dot(w0_ref[...], zl_ref[...], preferred_element_type=jnp.float32)
         + jnp.dot(w1_ref[...], zc_ref[...], preferred_element_type=jnp.float32)
         + jnp.dot(w2_ref[...], zr_ref[...], preferred_element_type=jnp.float32)
         + cb_ref[...])
    z = jnp.maximum(z, 0.0)
    col = i * _CB + lax.broadcasted_iota(jnp.int32, (32, _CB), 1)
    acc_ref[...] += jnp.where(col < N, z, 0.0)

    @pl.when(i == _NBLK - 1)
    def _fin():
        m = jnp.sum(acc_ref[...], axis=1, keepdims=True) * (1.0 / N)
        val = (jnp.sum(m * fcw_ref[...], keepdims=True).reshape(1, 1)
               + fcb_ref[...])
        out_ref[...] = 1.0 / (1.0 + jnp.exp(-val))


def _tc_conv(zc, zl, zr, w0, w1, w2, cb2, fcw2, fcb2):
    return pl.pallas_call(
        _tc_conv_body,
        grid=(_NBLK,),
        in_specs=[
            pl.BlockSpec((16, _CB), lambda i: (0, i)),
            pl.BlockSpec((16, _CB), lambda i: (0, i)),
            pl.BlockSpec((16, _CB), lambda i: (0, i)),
            pl.BlockSpec((32, 16), lambda i: (0, 0)),
            pl.BlockSpec((32, 16), lambda i: (0, 0)),
            pl.BlockSpec((32, 16), lambda i: (0, 0)),
            pl.BlockSpec((32, 1), lambda i: (0, 0)),
            pl.BlockSpec((32, 1), lambda i: (0, 0)),
            pl.BlockSpec((1, 1), lambda i: (0, 0)),
        ],
        out_specs=pl.BlockSpec((1, 1), lambda i: (0, 0)),
        out_shape=jax.ShapeDtypeStruct((1, 1), jnp.float32),
        scratch_shapes=[pltpu.VMEM((32, _CB), jnp.float32)],
    )(zc, zl, zr, w0, w1, w2, cb2, fcw2, fcb2)


def kernel(x, edge_index, W_gcn, b_gcn, conv_w, conv_b, fc_w, fc_b):
    src = edge_index[0]
    dst = edge_index[1]

    ones4 = jnp.ones((WIN, 4), jnp.float32)
    zeros4 = jnp.zeros((RPT, 4), jnp.float32)
    deg4 = _sc_hist(dst, ones4, zeros4)
    deg4r = deg4.reshape(NC, FROWS, 128)
    x_flat = jnp.pad(x, ((0, NP - N), (0, 0))).reshape(FROWS, 128)

    p_flat, dinv_flat = _tc_pre(x_flat, deg4r)

    p4 = p_flat.reshape(NP, 4)
    s4 = _sc_msg(src, dst, p4, zeros4)
    s4r = s4.reshape(NC, FROWS, 128)

    bigw = jnp.kron(jnp.eye(32, dtype=jnp.float32), W_gcn)
    bias_flat = jnp.tile(b_gcn, 32).reshape(1, 512)
    out_flat = _tc_gcn(s4r, p_flat, dinv_flat, bigw, bias_flat)

    z = out_flat[:N // 32].reshape(16, N)
    zc = jnp.pad(z, ((0, 0), (0, _LPAD - N)))
    zl = jnp.pad(z[:, :N - 1], ((0, 0), (1, _LPAD - N)))
    zr = jnp.pad(z[:, 1:], ((0, 0), (0, _LPAD - N + 1)))

    w0 = conv_w[:, :, 0]
    w1 = conv_w[:, :, 1]
    w2 = conv_w[:, :, 2]
    cb2 = conv_b.reshape(32, 1)
    fcw2 = fc_w.reshape(32, 1)
    fcb2 = fc_b.reshape(1, 1)

    y = _tc_conv(zc, zl, zr, w0, w1, w2, cb2, fcw2, fcb2)
    return y.reshape(-1)


# speedup vs baseline: 137.5287x; 1.5971x over previous
import jax
import jax.numpy as jnp
from jax import lax
from jax.experimental import pallas as pl
from jax.experimental.pallas import tpu as pltpu
from jax.experimental.pallas import tpu_sc as plsc

N = 100000
NP = 102400
E = 3200000
NC = 2
NS = 16
NW = NC * NS
PER_TILE = E // NW
WIN = 2000
NWIN = PER_TILE // WIN
RPT = NP // NS
FR = RPT * 4 // 128
FROWS = NP * 4 // 128
ZR = 40

_SC_MESH = plsc.VectorSubcoreMesh(core_axis_name="c", subcore_axis_name="s")
_SC_PARAMS = pltpu.CompilerParams(use_tc_tiling_on_sc=False,
                                  needs_layout_passes=False)


def _sc_hist_body(dst_hbm, ones_hbm, zeros_hbm, out_hbm, idx_v, ones_v,
                  deg_sh, sem):
    c = lax.axis_index("c")
    s = lax.axis_index("s")
    wid = c * NS + s
    pltpu.sync_copy(ones_hbm, ones_v)
    pltpu.sync_copy(zeros_hbm, deg_sh.at[pl.ds(s * RPT, RPT), :])
    plsc.subcore_barrier()
    base_e = wid * PER_TILE

    def body(w, carry):
        pltpu.sync_copy(dst_hbm.at[pl.ds(base_e + w * WIN, WIN)], idx_v)
        pltpu.sync_copy(ones_v, deg_sh.at[idx_v], add=True)
        return carry

    lax.fori_loop(0, NWIN, body, 0)
    plsc.subcore_barrier()
    pltpu.sync_copy(deg_sh.at[pl.ds(s * RPT, RPT), :],
                    out_hbm.at[c, pl.ds(s * RPT, RPT), :])


_sc_hist = pl.kernel(
    _sc_hist_body,
    out_type=jax.ShapeDtypeStruct((NC, NP, 4), jnp.float32),
    mesh=_SC_MESH,
    compiler_params=_SC_PARAMS,
    scratch_types=[
        pltpu.VMEM((WIN,), jnp.int32),
        pltpu.VMEM((WIN, 4), jnp.float32),
        pltpu.VMEM_SHARED((NP, 4), jnp.float32),
        pltpu.SemaphoreType.DMA,
    ],
)


CH = 800
CF = CH * 4 // 128
NCH = RPT // CH


def _rsqrt16(d):
    i = plsc.bitcast(d, jnp.int32)
    y = plsc.bitcast(jnp.int32(0x5F3759DF) - (i >> 1), jnp.float32)
    for _ in range(3):
        y = y * (1.5 - 0.5 * d * y * y)
    return y


def _sc_msg_body(src_hbm, dst_hbm, deg_hbm, x_hbm, zeros_hbm, out_hbm,
                 src_v, dst_v, rows_v, xb, vb, da, db, pb, dvb, p_sh,
                 acc_sh, sem):
    c = lax.axis_index("c")
    s = lax.axis_index("s")
    wid = c * NS + s
    row0 = s * RPT
    f0 = s * FR
    iota = lax.iota(jnp.int32, 16)
    r0 = iota >> 2
    c0 = iota & 3

    pltpu.sync_copy(zeros_hbm, acc_sh.at[pl.ds(row0, RPT), :])

    def prologue(ck, carry):
        pltpu.sync_copy(deg_hbm.at[0, pl.ds(row0 + ck * CH, CH), :], da)
        pltpu.sync_copy(deg_hbm.at[1, pl.ds(row0 + ck * CH, CH), :], db)
        pltpu.sync_copy(x_hbm.at[pl.ds(f0 + ck * CF, CF), :], xb)

        def group(g, carry2):
            rows = r0 + g * 4
            d0 = plsc.load_gather(da, [rows, c0])
            d1 = plsc.load_gather(db, [rows, c0])
            d = d0 + d1 + 1.0
            y = _rsqrt16(d)
            xv = xb[g >> 3, pl.ds((g & 7) * 16, 16)]
            p16 = y * xv
            plsc.store_scatter(pb, [rows, c0], p16)
            nl = (ck * CH + g * 4) + r0
            plsc.store_scatter(dvb, [nl >> 7, nl & 127], y, mask=(c0 == 0))
            return carry2

        lax.fori_loop(0, CH // 4, group, 0)
        pltpu.sync_copy(pb, p_sh.at[pl.ds(row0 + ck * CH, CH), :])
        return carry

    lax.fori_loop(0, NCH, prologue, 0)
    plsc.subcore_barrier()
    base_e = wid * PER_TILE

    def body(w, carry):
        pltpu.sync_copy(src_hbm.at[pl.ds(base_e + w * WIN, WIN)], src_v)
        pltpu.sync_copy(dst_hbm.at[pl.ds(base_e + w * WIN, WIN)], dst_v)
        pltpu.async_copy(p_sh.at[src_v], rows_v, sem).wait()
        pltpu.sync_copy(rows_v, acc_sh.at[dst_v], add=True)
        return carry

    lax.fori_loop(0, NWIN, body, 0)
    plsc.subcore_barrier()

    def drain(ck, carry):
        pltpu.sync_copy(acc_sh.at[pl.ds(row0 + ck * CH, CH), :], da)
        pltpu.sync_copy(x_hbm.at[pl.ds(f0 + ck * CF, CF), :], xb)

        def group(g, carry2):
            rows = r0 + g * 4
            a16 = plsc.load_gather(da, [rows, c0])
            nl = (ck * CH + g * 4) + r0
            y = plsc.load_gather(dvb, [nl >> 7, nl & 127])
            xv = xb[g >> 3, pl.ds((g & 7) * 16, 16)]
            v16 = y * a16 + 0.5 * y * y * xv
            vb[g >> 3, pl.ds((g & 7) * 16, 16)] = v16
            return carry2

        lax.fori_loop(0, CH // 4, group, 0)
        pltpu.sync_copy(vb, out_hbm.at[c, pl.ds(f0 + ck * CF, CF), :])
        return carry

    lax.fori_loop(0, NCH, drain, 0)


_sc_msg = pl.kernel(
    _sc_msg_body,
    out_type=jax.ShapeDtypeStruct((NC, FROWS, 128), jnp.float32),
    mesh=_SC_MESH,
    compiler_params=_SC_PARAMS,
    scratch_types=[
        pltpu.VMEM((WIN,), jnp.int32),
        pltpu.VMEM((WIN,), jnp.int32),
        pltpu.VMEM((WIN, 4), jnp.float32),
        pltpu.VMEM((CF, 128), jnp.float32),
        pltpu.VMEM((CF, 128), jnp.float32),
        pltpu.VMEM((CH, 4), jnp.float32),
        pltpu.VMEM((CH, 4), jnp.float32),
        pltpu.VMEM((CH, 4), jnp.float32),
        pltpu.VMEM((RPT // 128, 128), jnp.float32),
        pltpu.VMEM_SHARED((NP, 4), jnp.float32),
        pltpu.VMEM_SHARED((NP, 4), jnp.float32),
        pltpu.SemaphoreType.DMA,
    ],
)


def _tc_gcn_body(v_ref, bigw_ref, bias_ref, out_ref):
    u = v_ref[0] + v_ref[1]
    t = jnp.dot(u, bigw_ref[...], preferred_element_type=jnp.float32)
    out_ref[...] = jnp.maximum(t + bias_ref[...], 0.0)


def _tc_gcn(v, bigw, bias_flat):
    return pl.pallas_call(
        _tc_gcn_body,
        out_shape=jax.ShapeDtypeStruct((FROWS, 512), jnp.float32),
    )(v, bigw, bias_flat)


_CB = 1024
_LPAD = 100352
_NBLK = _LPAD // _CB


def _tc_conv_body(zc_ref, zl_ref, zr_ref, w0_ref, w1_ref, w2_ref, cb_ref,
                  fcw_ref, fcb_ref, out_ref, acc_ref):
    i = pl.program_id(0)

    @pl.when(i == 0)
    def _init():
        acc_ref[...] = jnp.zeros_like(acc_ref)

    z = (jnp.dot(w0_ref[...], zl_ref[...], preferred_element_type=jnp.float32)
         + jnp.dot(w1_ref[...], zc_ref[...], preferred_element_type=jnp.float32)
         + jnp.dot(w2_ref[...], zr_ref[...], preferred_element_type=jnp.float32)
         + cb_ref[...])
    z = jnp.maximum(z, 0.0)
    col = i * _CB + lax.broadcasted_iota(jnp.int32, (32, _CB), 1)
    acc_ref[...] += jnp.where(col < N, z, 0.0)

    @pl.when(i == _NBLK - 1)
    def _fin():
        m = jnp.sum(acc_ref[...], axis=1, keepdims=True) * (1.0 / N)
        val = (jnp.sum(m * fcw_ref[...], keepdims=True).reshape(1, 1)
               + fcb_ref[...])
        out_ref[...] = 1.0 / (1.0 + jnp.exp(-val))


def _tc_conv(zc, zl, zr, w0, w1, w2, cb2, fcw2, fcb2):
    return pl.pallas_call(
        _tc_conv_body,
        grid=(_NBLK,),
        in_specs=[
            pl.BlockSpec((16, _CB), lambda i: (0, i)),
            pl.BlockSpec((16, _CB), lambda i: (0, i)),
            pl.BlockSpec((16, _CB), lambda i: (0, i)),
            pl.BlockSpec((32, 16), lambda i: (0, 0)),
            pl.BlockSpec((32, 16), lambda i: (0, 0)),
            pl.BlockSpec((32, 16), lambda i: (0, 0)),
            pl.BlockSpec((32, 1), lambda i: (0, 0)),
            pl.BlockSpec((32, 1), lambda i: (0, 0)),
            pl.BlockSpec((1, 1), lambda i: (0, 0)),
        ],
        out_specs=pl.BlockSpec((1, 1), lambda i: (0, 0)),
        out_shape=jax.ShapeDtypeStruct((1, 1), jnp.float32),
        scratch_shapes=[pltpu.VMEM((32, _CB), jnp.float32)],
    )(zc, zl, zr, w0, w1, w2, cb2, fcw2, fcb2)


def kernel(x, edge_index, W_gcn, b_gcn, conv_w, conv_b, fc_w, fc_b):
    src = edge_index[0]
    dst = edge_index[1]

    ones4 = jnp.ones((WIN, 4), jnp.float32)
    zeros4 = jnp.zeros((RPT, 4), jnp.float32)
    deg4 = _sc_hist(dst, ones4, zeros4)
    x_flat = jnp.pad(x, ((0, NP - N), (0, 0))).reshape(FROWS, 128)

    v = _sc_msg(src, dst, deg4, x_flat, zeros4)

    bigw = jnp.kron(jnp.eye(32, dtype=jnp.float32), W_gcn)
    bias_flat = jnp.tile(b_gcn, 32).reshape(1, 512)
    out_flat = _tc_gcn(v, bigw, bias_flat)

    z = out_flat[:N // 32].reshape(16, N)
    zc = jnp.pad(z, ((0, 0), (0, _LPAD - N)))
    zl = jnp.pad(z[:, :N - 1], ((0, 0), (1, _LPAD - N)))
    zr = jnp.pad(z[:, 1:], ((0, 0), (0, _LPAD - N + 1)))

    w0 = conv_w[:, :, 0]
    w1 = conv_w[:, :, 1]
    w2 = conv_w[:, :, 2]
    cb2 = conv_b.reshape(32, 1)
    fcw2 = fc_w.reshape(32, 1)
    fcb2 = fc_b.reshape(1, 1)

    y = _tc_conv(zc, zl, zr, w0, w1, w2, cb2, fcw2, fcb2)
    return y.reshape(-1)


# speedup vs baseline: 144.1673x; 1.0483x over previous
import jax
import jax.numpy as jnp
from jax import lax
from jax.experimental import pallas as pl
from jax.experimental.pallas import tpu as pltpu
from jax.experimental.pallas import tpu_sc as plsc

N = 100000
NP = 102400
E = 3200000
NC = 2
NS = 16
NW = NC * NS
PER_TILE = E // NW
WIN = 2000
NWIN = PER_TILE // WIN
RPT = NP // NS
FR = RPT * 4 // 128
FROWS = NP * 4 // 128
CH = 800
CF = CH * 4 // 128
NCH = RPT // CH

_SC_MESH = plsc.VectorSubcoreMesh(core_axis_name="c", subcore_axis_name="s")
_SC_PARAMS = pltpu.CompilerParams(use_tc_tiling_on_sc=False,
                                  needs_layout_passes=False)


def _fill4(ref, nrows, val):
    iota = lax.iota(jnp.int32, 16)
    r0 = iota >> 2
    c0 = iota & 3
    v = jnp.full((16,), val, jnp.float32)

    def body(g, carry):
        plsc.store_scatter(ref, [r0 + g * 4, c0], v)
        return carry

    lax.fori_loop(0, nrows // 4, body, 0)


HW = 2000
HK = PER_TILE // (2 * HW)


def _sc_hist_body(dst_hbm, out_hbm, ia, ib, ones_v, zb, deg_sh,
                  sia, sib, ssa, ssb):
    c = lax.axis_index("c")
    s = lax.axis_index("s")
    wid = c * NS + s
    _fill4(ones_v, HW, 1.0)
    _fill4(zb, CH, 0.0)
    for k in range(RPT // CH):
        pltpu.sync_copy(zb, deg_sh.at[pl.ds(s * RPT + k * CH, CH), :])
    plsc.subcore_barrier()
    base_e = wid * PER_TILE

    def win(w):
        return dst_hbm.at[pl.ds(base_e + w * HW, HW)]

    def body(k, carry):
        w = 2 * k
        pltpu.sync_copy(win(w), ia)
        pltpu.sync_copy(ones_v, deg_sh.at[ia], add=True)
        pltpu.sync_copy(win(w + 1), ib)
        pltpu.sync_copy(ones_v, deg_sh.at[ib], add=True)
        return carry

    lax.fori_loop(0, HK, body, 0)
    plsc.subcore_barrier()
    pltpu.sync_copy(deg_sh.at[pl.ds(s * RPT, RPT), :],
                    out_hbm.at[c, pl.ds(s * RPT, RPT), :])


_sc_hist = pl.kernel(
    _sc_hist_body,
    out_type=jax.ShapeDtypeStruct((NC, NP, 4), jnp.float32),
    mesh=_SC_MESH,
    compiler_params=_SC_PARAMS,
    scratch_types=[
        pltpu.VMEM((HW,), jnp.int32),
        pltpu.VMEM((HW,), jnp.int32),
        pltpu.VMEM((HW, 4), jnp.float32),
        pltpu.VMEM((CH, 4), jnp.float32),
        pltpu.VMEM_SHARED((NP, 4), jnp.float32),
        pltpu.SemaphoreType.DMA,
        pltpu.SemaphoreType.DMA,
        pltpu.SemaphoreType.DMA,
        pltpu.SemaphoreType.DMA,
    ],
)


def _rsqrt16(d):
    i = plsc.bitcast(d, jnp.int32)
    y = plsc.bitcast(jnp.int32(0x5F3759DF) - (i >> 1), jnp.float32)
    for _ in range(3):
        y = y * (1.5 - 0.5 * d * y * y)
    return y


def _sc_msg_body(src_hbm, dst_hbm, deg_hbm, x_hbm, out_hbm,
                 sa_v, da_v, sb_v, db_v, ra_v, rb_v, xb, vb, da, db, pb,
                 dvb, p_sh, acc_sh, sia, sib, sga, sgb, ssa, ssb):
    c = lax.axis_index("c")
    s = lax.axis_index("s")
    wid = c * NS + s
    row0 = s * RPT
    f0 = s * FR
    iota = lax.iota(jnp.int32, 16)
    r0 = iota >> 2
    c0 = iota & 3

    _fill4(pb, CH, 0.0)
    for k in range(RPT // CH):
        pltpu.sync_copy(pb, acc_sh.at[pl.ds(row0 + k * CH, CH), :])

    def prologue(ck, carry):
        pltpu.sync_copy(deg_hbm.at[0, pl.ds(row0 + ck * CH, CH), :], da)
        pltpu.sync_copy(deg_hbm.at[1, pl.ds(row0 + ck * CH, CH), :], db)
        pltpu.sync_copy(x_hbm.at[pl.ds(f0 + ck * CF, CF), :], xb)

        def group(g, carry2):
            rows = r0 + g * 4
            d0 = plsc.load_gather(da, [rows, c0])
            d1 = plsc.load_gather(db, [rows, c0])
            d = d0 + d1 + 1.0
            y = _rsqrt16(d)
            xv = xb[g >> 3, pl.ds((g & 7) * 16, 16)]
            p16 = y * xv
            plsc.store_scatter(pb, [rows, c0], p16)
            nl = (ck * CH + g * 4) + r0
            plsc.store_scatter(dvb, [nl >> 7, nl & 127], y, mask=(c0 == 0))
            return carry2

        lax.fori_loop(0, CH // 4, group, 0)
        pltpu.sync_copy(pb, p_sh.at[pl.ds(row0 + ck * CH, CH), :])
        return carry

    lax.fori_loop(0, NCH, prologue, 0)
    plsc.subcore_barrier()
    base_e = wid * PER_TILE

    def wsrc(w):
        return src_hbm.at[pl.ds(base_e + w * WIN, WIN)]

    def wdst(w):
        return dst_hbm.at[pl.ds(base_e + w * WIN, WIN)]

    def body(w, carry):
        pltpu.sync_copy(wsrc(w), sa_v)
        pltpu.sync_copy(wdst(w), da_v)
        pltpu.async_copy(p_sh.at[sa_v], ra_v, sga).wait()
        pltpu.sync_copy(ra_v, acc_sh.at[da_v], add=True)
        return carry

    lax.fori_loop(0, NWIN, body, 0)
    plsc.subcore_barrier()

    def drain(ck, carry):
        pltpu.sync_copy(acc_sh.at[pl.ds(row0 + ck * CH, CH), :], da)
        pltpu.sync_copy(x_hbm.at[pl.ds(f0 + ck * CF, CF), :], xb)

        def group(g, carry2):
            rows = r0 + g * 4
            a16 = plsc.load_gather(da, [rows, c0])
            nl = (ck * CH + g * 4) + r0
            y = plsc.load_gather(dvb, [nl >> 7, nl & 127])
            xv = xb[g >> 3, pl.ds((g & 7) * 16, 16)]
            v16 = y * a16 + 0.5 * y * y * xv
            vb[g >> 3, pl.ds((g & 7) * 16, 16)] = v16
            return carry2

        lax.fori_loop(0, CH // 4, group, 0)
        pltpu.sync_copy(vb, out_hbm.at[c, pl.ds(f0 + ck * CF, CF), :])
        return carry

    lax.fori_loop(0, NCH, drain, 0)


_sc_msg = pl.kernel(
    _sc_msg_body,
    out_type=jax.ShapeDtypeStruct((NC, FROWS, 128), jnp.float32),
    mesh=_SC_MESH,
    compiler_params=_SC_PARAMS,
    scratch_types=[
        pltpu.VMEM((WIN,), jnp.int32),
        pltpu.VMEM((WIN,), jnp.int32),
        pltpu.VMEM((WIN,), jnp.int32),
        pltpu.VMEM((WIN,), jnp.int32),
        pltpu.VMEM((WIN, 4), jnp.float32),
        pltpu.VMEM((WIN, 4), jnp.float32),
        pltpu.VMEM((CF, 128), jnp.float32),
        pltpu.VMEM((CF, 128), jnp.float32),
        pltpu.VMEM((CH, 4), jnp.float32),
        pltpu.VMEM((CH, 4), jnp.float32),
        pltpu.VMEM((CH, 4), jnp.float32),
        pltpu.VMEM((RPT // 128, 128), jnp.float32),
        pltpu.VMEM_SHARED((NP, 4), jnp.float32),
        pltpu.VMEM_SHARED((NP, 4), jnp.float32),
        pltpu.SemaphoreType.DMA,
        pltpu.SemaphoreType.DMA,
        pltpu.SemaphoreType.DMA,
        pltpu.SemaphoreType.DMA,
        pltpu.SemaphoreType.DMA,
        pltpu.SemaphoreType.DMA,
    ],
)


def _tc_gcn_body(v_ref, bigw_ref, bias_ref, out_ref):
    u = v_ref[0] + v_ref[1]
    t = jnp.dot(u, bigw_ref[...], preferred_element_type=jnp.float32)
    out_ref[...] = jnp.maximum(t + bias_ref[...], 0.0)


def _tc_gcn(v, bigw, bias_flat):
    return pl.pallas_call(
        _tc_gcn_body,
        out_shape=jax.ShapeDtypeStruct((FROWS, 512), jnp.float32),
    )(v, bigw, bias_flat)


_CB = 1024
_LPAD = 100352
_NBLK = _LPAD // _CB


def _tc_conv_body(zc_ref, zl_ref, zr_ref, w0_ref, w1_ref, w2_ref, cb_ref,
                  fcw_ref, fcb_ref, out_ref, acc_ref):
    i = pl.program_id(0)

    @pl.when(i == 0)
    def _init():
        acc_ref[...] = jnp.zeros_like(acc_ref)

    z = (jnp.dot(w0_ref[...], zl_ref[...], preferred_element_type=jnp.float32)
         + jnp.dot(w1_ref[...], zc_ref[...], preferred_element_type=jnp.float32)
         + jnp.dot(w2_ref[...], zr_ref[...], preferred_element_type=jnp.float32)
         + cb_ref[...])
    z = jnp.maximum(z, 0.0)
    col = i * _CB + lax.broadcasted_iota(jnp.int32, (32, _CB), 1)
    acc_ref[...] += jnp.where(col < N, z, 0.0)

    @pl.when(i == _NBLK - 1)
    def _fin():
        m = jnp.sum(acc_ref[...], axis=1, keepdims=True) * (1.0 / N)
        val = (jnp.sum(m * fcw_ref[...], keepdims=True).reshape(1, 1)
               + fcb_ref[...])
        out_ref[...] = 1.0 / (1.0 + jnp.exp(-val))


def _tc_conv(zc, zl, zr, w0, w1, w2, cb2, fcw2, fcb2):
    return pl.pallas_call(
        _tc_conv_body,
        grid=(_NBLK,),
        in_specs=[
            pl.BlockSpec((16, _CB), lambda i: (0, i)),
            pl.BlockSpec((16, _CB), lambda i: (0, i)),
            pl.BlockSpec((16, _CB), lambda i: (0, i)),
            pl.BlockSpec((32, 16), lambda i: (0, 0)),
            pl.BlockSpec((32, 16), lambda i: (0, 0)),
            pl.BlockSpec((32, 16), lambda i: (0, 0)),
            pl.BlockSpec((32, 1), lambda i: (0, 0)),
            pl.BlockSpec((32, 1), lambda i: (0, 0)),
            pl.BlockSpec((1, 1), lambda i: (0, 0)),
        ],
        out_specs=pl.BlockSpec((1, 1), lambda i: (0, 0)),
        out_shape=jax.ShapeDtypeStruct((1, 1), jnp.float32),
        scratch_shapes=[pltpu.VMEM((32, _CB), jnp.float32)],
    )(zc, zl, zr, w0, w1, w2, cb2, fcw2, fcb2)


def kernel(x, edge_index, W_gcn, b_gcn, conv_w, conv_b, fc_w, fc_b):
    src = edge_index[0]
    dst = edge_index[1]

    deg4 = _sc_hist(dst)
    x_flat = jnp.pad(x, ((0, NP - N), (0, 0))).reshape(FROWS, 128)

    v = _sc_msg(src, dst, deg4, x_flat)

    bigw = jnp.kron(jnp.eye(32, dtype=jnp.float32), W_gcn)
    bias_flat = jnp.tile(b_gcn, 32).reshape(1, 512)
    out_flat = _tc_gcn(v, bigw, bias_flat)

    z = out_flat[:N // 32].reshape(16, N)
    zc = jnp.pad(z, ((0, 0), (0, _LPAD - N)))
    zl = jnp.pad(z[:, :N - 1], ((0, 0), (1, _LPAD - N)))
    zr = jnp.pad(z[:, 1:], ((0, 0), (0, _LPAD - N + 1)))

    w0 = conv_w[:, :, 0]
    w1 = conv_w[:, :, 1]
    w2 = conv_w[:, :, 2]
    cb2 = conv_b.reshape(32, 1)
    fcw2 = fc_w.reshape(32, 1)
    fcb2 = fc_b.reshape(1, 1)

    y = _tc_conv(zc, zl, zr, w0, w1, w2, cb2, fcw2, fcb2)
    return y.reshape(-1)


# speedup vs baseline: 152.0230x; 1.0545x over previous
import jax
import jax.numpy as jnp
from jax import lax
from jax.experimental import pallas as pl
from jax.experimental.pallas import tpu as pltpu
from jax.experimental.pallas import tpu_sc as plsc

N = 100000
NP = 102400
E = 3200000
NC = 2
NS = 16
NW = NC * NS
PER_TILE = E // NW
WIN = 2000
NWIN = PER_TILE // WIN
RPT = NP // NS
FR = RPT * 4 // 128
FROWS = NP * 4 // 128
CH = 800
CF = CH * 4 // 128
NCH = RPT // CH

_SC_MESH = plsc.VectorSubcoreMesh(core_axis_name="c", subcore_axis_name="s")
_SC_PARAMS = pltpu.CompilerParams(use_tc_tiling_on_sc=False,
                                  needs_layout_passes=False)


def _fill4(ref, nrows, val):
    iota = lax.iota(jnp.int32, 16)
    r0 = iota >> 2
    c0 = iota & 3
    v = jnp.full((16,), val, jnp.float32)

    def body(g, carry):
        plsc.store_scatter(ref, [r0 + g * 4, c0], v)
        return carry

    lax.fori_loop(0, nrows // 4, body, 0)


HW = 2000
HK = PER_TILE // (2 * HW)


def _sc_hist_body(dst_hbm, out_hbm, ia, ib, ones_v, zb, deg_sh, sia):
    c = lax.axis_index("c")
    s = lax.axis_index("s")
    wid = c * NS + s
    _fill4(ones_v, HW, 1.0)
    _fill4(zb, CH, 0.0)
    for k in range(RPT // CH):
        pltpu.sync_copy(zb, deg_sh.at[pl.ds(s * RPT + k * CH, CH), :])
    plsc.subcore_barrier()
    base_e = wid * PER_TILE

    def win(w):
        return dst_hbm.at[pl.ds(base_e + w * HW, HW)]

    def body(k, carry):
        w = 2 * k
        pltpu.sync_copy(win(w), ia)
        pltpu.sync_copy(ones_v, deg_sh.at[ia], add=True)
        pltpu.sync_copy(win(w + 1), ib)
        pltpu.sync_copy(ones_v, deg_sh.at[ib], add=True)
        return carry

    lax.fori_loop(0, HK, body, 0)
    plsc.subcore_barrier()
    pltpu.sync_copy(deg_sh.at[pl.ds(s * RPT, RPT), :],
                    out_hbm.at[c, pl.ds(s * RPT, RPT), :])


_sc_hist = pl.kernel(
    _sc_hist_body,
    out_type=jax.ShapeDtypeStruct((NC, NP, 4), jnp.float32),
    mesh=_SC_MESH,
    compiler_params=_SC_PARAMS,
    scratch_types=[
        pltpu.VMEM((HW,), jnp.int32),
        pltpu.VMEM((HW,), jnp.int32),
        pltpu.VMEM((HW, 4), jnp.float32),
        pltpu.VMEM((CH, 4), jnp.float32),
        pltpu.VMEM_SHARED((NP, 4), jnp.float32),
        pltpu.SemaphoreType.DMA,
    ],
)


def _rsqrt16(d):
    i = plsc.bitcast(d, jnp.int32)
    y = plsc.bitcast(jnp.int32(0x5F3759DF) - (i >> 1), jnp.float32)
    for _ in range(3):
        y = y * (1.5 - 0.5 * d * y * y)
    return y


def _sc_msg_body(src_hbm, dst_hbm, deg_hbm, x_hbm, out_hbm,
                 sa_v, da_v, ra_v, xb, vb, da, db, pb,
                 dvb, p_sh, acc_sh, sga):
    c = lax.axis_index("c")
    s = lax.axis_index("s")
    wid = c * NS + s
    row0 = s * RPT
    f0 = s * FR
    iota = lax.iota(jnp.int32, 16)
    r0 = iota >> 2
    c0 = iota & 3

    _fill4(pb, CH, 0.0)
    for k in range(RPT // CH):
        pltpu.sync_copy(pb, acc_sh.at[pl.ds(row0 + k * CH, CH), :])

    def prologue(ck, carry):
        pltpu.sync_copy(deg_hbm.at[0, pl.ds(row0 + ck * CH, CH), :], da)
        pltpu.sync_copy(deg_hbm.at[1, pl.ds(row0 + ck * CH, CH), :], db)
        pltpu.sync_copy(x_hbm.at[pl.ds(f0 + ck * CF, CF), :], xb)

        def group(g, carry2):
            rows = r0 + g * 4
            d0 = plsc.load_gather(da, [rows, c0])
            d1 = plsc.load_gather(db, [rows, c0])
            d = d0 + d1 + 1.0
            y = _rsqrt16(d)
            xv = xb[g >> 3, pl.ds((g & 7) * 16, 16)]
            p16 = y * xv
            plsc.store_scatter(pb, [rows, c0], p16)
            nl = (ck * CH + g * 4) + r0
            plsc.store_scatter(dvb, [nl >> 7, nl & 127], y, mask=(c0 == 0))
            return carry2

        lax.fori_loop(0, CH // 4, group, 0)
        pltpu.sync_copy(pb, p_sh.at[pl.ds(row0 + ck * CH, CH), :])
        return carry

    lax.fori_loop(0, NCH, prologue, 0)
    plsc.subcore_barrier()
    base_e = wid * PER_TILE

    def wsrc(w):
        return src_hbm.at[pl.ds(base_e + w * WIN, WIN)]

    def wdst(w):
        return dst_hbm.at[pl.ds(base_e + w * WIN, WIN)]

    def body(w, carry):
        pltpu.sync_copy(wsrc(w), sa_v)
        pltpu.sync_copy(wdst(w), da_v)
        pltpu.async_copy(p_sh.at[sa_v], ra_v, sga).wait()
        pltpu.sync_copy(ra_v, acc_sh.at[da_v], add=True)
        return carry

    lax.fori_loop(0, NWIN, body, 0)
    plsc.subcore_barrier()

    def drain(ck, carry):
        pltpu.sync_copy(acc_sh.at[pl.ds(row0 + ck * CH, CH), :], da)
        pltpu.sync_copy(x_hbm.at[pl.ds(f0 + ck * CF, CF), :], xb)

        def group(g, carry2):
            rows = r0 + g * 4
            a16 = plsc.load_gather(da, [rows, c0])
            nl = (ck * CH + g * 4) + r0
            y = plsc.load_gather(dvb, [nl >> 7, nl & 127])
            xv = xb[g >> 3, pl.ds((g & 7) * 16, 16)]
            v16 = y * a16 + 0.5 * y * y * xv
            vb[g >> 3, pl.ds((g & 7) * 16, 16)] = v16
            return carry2

        lax.fori_loop(0, CH // 4, group, 0)
        pltpu.sync_copy(vb, out_hbm.at[c, pl.ds(f0 + ck * CF, CF), :])
        return carry

    lax.fori_loop(0, NCH, drain, 0)


_sc_msg = pl.kernel(
    _sc_msg_body,
    out_type=jax.ShapeDtypeStruct((NC, FROWS, 128), jnp.float32),
    mesh=_SC_MESH,
    compiler_params=_SC_PARAMS,
    scratch_types=[
        pltpu.VMEM((WIN,), jnp.int32),
        pltpu.VMEM((WIN,), jnp.int32),
        pltpu.VMEM((WIN, 4), jnp.float32),
        pltpu.VMEM((CF, 128), jnp.float32),
        pltpu.VMEM((CF, 128), jnp.float32),
        pltpu.VMEM((CH, 4), jnp.float32),
        pltpu.VMEM((CH, 4), jnp.float32),
        pltpu.VMEM((CH, 4), jnp.float32),
        pltpu.VMEM((RPT // 128, 128), jnp.float32),
        pltpu.VMEM_SHARED((NP, 4), jnp.float32),
        pltpu.VMEM_SHARED((NP, 4), jnp.float32),
        pltpu.SemaphoreType.DMA,
    ],
)


def _tc_gcn_body(v_ref, bigw_ref, bias_ref, out_ref):
    u = v_ref[0] + v_ref[1]
    t = jnp.dot(u, bigw_ref[...], preferred_element_type=jnp.float32)
    out_ref[...] = jnp.maximum(t + bias_ref[...], 0.0)


def _tc_gcn(v, bigw, bias_flat):
    return pl.pallas_call(
        _tc_gcn_body,
        out_shape=jax.ShapeDtypeStruct((FROWS, 512), jnp.float32),
    )(v, bigw, bias_flat)


_CB = 2048
_LPAD = 100352
_NBLK = _LPAD // _CB


def _tc_conv_body(zc_ref, zl_ref, zr_ref, w0_ref, w1_ref, w2_ref, cb_ref,
                  fcw_ref, fcb_ref, out_ref, acc_ref):
    i = pl.program_id(0)

    @pl.when(i == 0)
    def _init():
        acc_ref[...] = jnp.zeros_like(acc_ref)

    z = (jnp.dot(w0_ref[...], zl_ref[...], preferred_element_type=jnp.float32)
         + jnp.dot(w1_ref[...], zc_ref[...], preferred_element_type=jnp.float32)
         + jnp.dot(w2_ref[...], zr_ref[...], preferred_element_type=jnp.float32)
         + cb_ref[...])
    z = jnp.maximum(z, 0.0)
    col = i * _CB + lax.broadcasted_iota(jnp.int32, (32, _CB), 1)
    acc_ref[...] += jnp.where(col < N, z, 0.0)

    @pl.when(i == _NBLK - 1)
    def _fin():
        m = jnp.sum(acc_ref[...], axis=1, keepdims=True) * (1.0 / N)
        val = (jnp.sum(m * fcw_ref[...], keepdims=True).reshape(1, 1)
               + fcb_ref[...])
        out_ref[...] = 1.0 / (1.0 + jnp.exp(-val))


def _tc_conv(zc, zl, zr, w0, w1, w2, cb2, fcw2, fcb2):
    return pl.pallas_call(
        _tc_conv_body,
        grid=(_NBLK,),
        in_specs=[
            pl.BlockSpec((16, _CB), lambda i: (0, i)),
            pl.BlockSpec((16, _CB), lambda i: (0, i)),
            pl.BlockSpec((16, _CB), lambda i: (0, i)),
            pl.BlockSpec((32, 16), lambda i: (0, 0)),
            pl.BlockSpec((32, 16), lambda i: (0, 0)),
            pl.BlockSpec((32, 16), lambda i: (0, 0)),
            pl.BlockSpec((32, 1), lambda i: (0, 0)),
            pl.BlockSpec((32, 1), lambda i: (0, 0)),
            pl.BlockSpec((1, 1), lambda i: (0, 0)),
        ],
        out_specs=pl.BlockSpec((1, 1), lambda i: (0, 0)),
        out_shape=jax.ShapeDtypeStruct((1, 1), jnp.float32),
        scratch_shapes=[pltpu.VMEM((32, _CB), jnp.float32)],
    )(zc, zl, zr, w0, w1, w2, cb2, fcw2, fcb2)


def kernel(x, edge_index, W_gcn, b_gcn, conv_w, conv_b, fc_w, fc_b):
    src = edge_index[0]
    dst = edge_index[1]

    deg4 = _sc_hist(dst)
    x_flat = jnp.pad(x, ((0, NP - N), (0, 0))).reshape(FROWS, 128)

    v = _sc_msg(src, dst, deg4, x_flat)

    bigw = jnp.kron(jnp.eye(32, dtype=jnp.float32), W_gcn)
    bias_flat = jnp.tile(b_gcn, 32).reshape(1, 512)
    out_flat = _tc_gcn(v, bigw, bias_flat)

    z = out_flat[:N // 32].reshape(16, N)
    zc = jnp.pad(z, ((0, 0), (0, _LPAD - N)))
    zl = jnp.pad(z[:, :N - 1], ((0, 0), (1, _LPAD - N)))
    zr = jnp.pad(z[:, 1:], ((0, 0), (0, _LPAD - N + 1)))

    w0 = conv_w[:, :, 0]
    w1 = conv_w[:, :, 1]
    w2 = conv_w[:, :, 2]
    cb2 = conv_b.reshape(32, 1)
    fcw2 = fc_w.reshape(32, 1)
    fcb2 = fc_b.reshape(1, 1)

    y = _tc_conv(zc, zl, zr, w0, w1, w2, cb2, fcw2, fcb2)
    return y.reshape(-1)
